# baseline (device time: 112694 ns/iter reference)
import jax
import jax.numpy as jnp
from jax import lax
from jax.experimental import pallas as pl
from jax.experimental.pallas import tpu as pltpu


def kernel(O, Wo):
    B, S, H, D = O.shape
    K = H * D
    N = Wo.shape[1]
    S_half = S // 2

    O2 = O.reshape(B, S, K)

    def body(o_ref, w_ref, out_ref, send_buf, recv_buf, send_sem, recv_sem):
        my_x = lax.axis_index("x")
        my_y = lax.axis_index("y")
        my_z = lax.axis_index("z")
        peer = (1 - my_x, my_y, my_z)

        barrier_sem = pltpu.get_barrier_semaphore()
        pl.semaphore_signal(
            barrier_sem, inc=1, device_id=peer,
            device_id_type=pl.DeviceIdType.MESH,
        )
        pl.semaphore_wait(barrier_sem, 1)

        my_start = my_x * S_half
        peer_start = (1 - my_x) * S_half
        w = w_ref[:, :]

        o_peer = o_ref[:, pl.ds(peer_start, S_half), :]
        send_buf[:, :, :] = lax.dot_general(
            o_peer, w, (((2,), (0,)), ((), ())),
            preferred_element_type=jnp.float32,
        )

        rdma = pltpu.make_async_remote_copy(
            src_ref=send_buf,
            dst_ref=recv_buf,
            send_sem=send_sem,
            recv_sem=recv_sem,
            device_id=peer,
            device_id_type=pl.DeviceIdType.MESH,
        )
        rdma.start()

        o_mine = o_ref[:, pl.ds(my_start, S_half), :]
        out_ref[:, :, :] = lax.dot_general(
            o_mine, w, (((2,), (0,)), ((), ())),
            preferred_element_type=jnp.float32,
        )

        rdma.wait()
        out_ref[:, :, :] += recv_buf[:, :, :]

    return pl.pallas_call(
        body,
        out_shape=jax.ShapeDtypeStruct((B, S_half, N), jnp.float32),
        in_specs=[
            pl.BlockSpec(memory_space=pltpu.VMEM),
            pl.BlockSpec(memory_space=pltpu.VMEM),
        ],
        out_specs=pl.BlockSpec(memory_space=pltpu.VMEM),
        scratch_shapes=[
            pltpu.VMEM((B, S_half, N), jnp.float32),
            pltpu.VMEM((B, S_half, N), jnp.float32),
            pltpu.SemaphoreType.DMA,
            pltpu.SemaphoreType.DMA,
        ],
        compiler_params=pltpu.CompilerParams(collective_id=0),
    )(O2, Wo)


# device time: 108023 ns/iter; 1.0432x vs baseline; 1.0432x over previous
import jax
import jax.numpy as jnp
from jax import lax
from jax.experimental import pallas as pl
from jax.experimental.pallas import tpu as pltpu

N_CHUNK = 8


def kernel(O, Wo):
    B, S, H, D = O.shape
    K = H * D
    N = Wo.shape[1]
    S_half = S // 2
    S_chunk = S_half // N_CHUNK

    O2 = O.reshape(B, S, K)

    def body(o_ref, w_ref, out_ref, send_buf, recv_buf, send_sems, recv_sems):
        my_x = lax.axis_index("x")
        my_y = lax.axis_index("y")
        my_z = lax.axis_index("z")
        peer = (1 - my_x, my_y, my_z)

        barrier_sem = pltpu.get_barrier_semaphore()
        pl.semaphore_signal(
            barrier_sem, inc=1, device_id=peer,
            device_id_type=pl.DeviceIdType.MESH,
        )
        pl.semaphore_wait(barrier_sem, 1)

        my_start = my_x * S_half
        peer_start = (1 - my_x) * S_half
        w = w_ref[:, :]

        def chunk_rdma(c):
            return pltpu.make_async_remote_copy(
                src_ref=send_buf.at[c],
                dst_ref=recv_buf.at[c],
                send_sem=send_sems.at[c],
                recv_sem=recv_sems.at[c],
                device_id=peer,
                device_id_type=pl.DeviceIdType.MESH,
            )

        rdmas = []
        for c in range(N_CHUNK):
            o_c = o_ref[:, pl.ds(peer_start + c * S_chunk, S_chunk), :]
            send_buf[c, :, :, :] = lax.dot_general(
                o_c, w, (((2,), (0,)), ((), ())),
                preferred_element_type=jnp.float32,
            )
            rdma = chunk_rdma(c)
            rdma.start()
            rdmas.append(rdma)

        o_mine = o_ref[:, pl.ds(my_start, S_half), :]
        out_ref[:, :, :] = lax.dot_general(
            o_mine, w, (((2,), (0,)), ((), ())),
            preferred_element_type=jnp.float32,
        )

        for c in range(N_CHUNK):
            rdmas[c].wait_recv()
            out_ref[:, pl.ds(c * S_chunk, S_chunk), :] += recv_buf[c, :, :, :]

        for c in range(N_CHUNK):
            rdmas[c].wait_send()

    return pl.pallas_call(
        body,
        out_shape=jax.ShapeDtypeStruct((B, S_half, N), jnp.float32),
        in_specs=[
            pl.BlockSpec(memory_space=pltpu.VMEM),
            pl.BlockSpec(memory_space=pltpu.VMEM),
        ],
        out_specs=pl.BlockSpec(memory_space=pltpu.VMEM),
        scratch_shapes=[
            pltpu.VMEM((N_CHUNK, B, S_chunk, N), jnp.float32),
            pltpu.VMEM((N_CHUNK, B, S_chunk, N), jnp.float32),
            pltpu.SemaphoreType.DMA((N_CHUNK,)),
            pltpu.SemaphoreType.DMA((N_CHUNK,)),
        ],
        compiler_params=pltpu.CompilerParams(collective_id=0),
    )(O2, Wo)
